# baseline (speedup 1.0000x reference)
import jax
import jax.numpy as jnp
from jax.experimental import pallas as pl

IN_DIM = 72
HIDDEN = 512
LATENT = 256
LEVELS = 2
K = 512
BETA = 0.25
BATCH = 4096
BB = 2048
NB = BATCH // BB
NCH = 1
CB = BB // NCH


def _fused_kernel(x_ref, we1_ref, be1_ref, we2_ref, be2_ref, cb0_ref, cb1_ref,
                  wd1_ref, bd1_ref, wd2_ref, bd2_ref,
                  recon_ref, idx0_ref, idx1_ref, sums_ref):
    i = pl.program_id(0)

    @pl.when(i == 0)
    def _init():
        sums_ref[...] = jnp.zeros_like(sums_ref)

    lanef = jax.lax.broadcasted_iota(
        jnp.int32, (CB, K), 1).astype(jnp.float32)

    def vq_level(residual, cb, exact_q):
        c2 = jnp.sum(cb * cb, axis=1)[None, :]
        r2 = jnp.sum(residual * residual, axis=1, keepdims=True)
        cross = 2.0 * jax.lax.dot_general(
            residual, cb, (((1,), (1,)), ((), ())),
            preferred_element_type=jnp.float32)
        d = (r2 - cross) + c2
        dmin = jnp.min(d, axis=1, keepdims=True)
        idxf = jnp.min(jnp.where(d == dmin, lanef, float(K)), axis=1,
                       keepdims=True)
        idx = idxf.astype(jnp.int32)
        oh = (lanef == idxf).astype(jnp.float32).astype(jnp.bfloat16)
        g = lambda m: jnp.dot(oh, m.astype(jnp.bfloat16),
                              preferred_element_type=jnp.float32)
        if exact_q:
            hi = cb.astype(jnp.bfloat16).astype(jnp.float32)
            r1 = cb - hi
            mid = r1.astype(jnp.bfloat16).astype(jnp.float32)
            lo = r1 - mid
            q = (g(hi) + g(mid)) + g(lo)
        else:
            q = g(cb)
        return idx, q

    rsum = 0.0
    qsum = 0.0
    for c in range(NCH):
        rows = slice(c * CB, (c + 1) * CB)
        x = x_ref[rows, :]
        h = jnp.maximum(
            jnp.dot(x, we1_ref[...], preferred_element_type=jnp.float32)
            + be1_ref[...], 0.0)
        latent = (jnp.dot(h, we2_ref[...], preferred_element_type=jnp.float32)
                  + be2_ref[...])
        idx0, q0 = vq_level(latent, cb0_ref[...], exact_q=True)
        residual = latent - q0
        idx1, q1 = vq_level(residual, cb1_ref[...], exact_q=False)
        quantized = q0 + q1
        qdiff = latent - quantized
        qst = latent + (quantized - latent)
        hd = jnp.maximum(
            jnp.dot(qst, wd1_ref[...], preferred_element_type=jnp.float32)
            + bd1_ref[...], 0.0)
        recon = (jnp.dot(hd, wd2_ref[...], preferred_element_type=jnp.float32)
                 + bd2_ref[...])

        recon_ref[rows, :] = recon
        idx0_ref[rows, :] = idx0
        idx1_ref[rows, :] = idx1
        rsum += jnp.sum((recon - x) ** 2)
        qsum += jnp.sum(qdiff * qdiff)

    lanev = jax.lax.broadcasted_iota(jnp.int32, sums_ref.shape, 1)
    sums_ref[...] += (jnp.where(lanev == 0, rsum, 0.0)
                      + jnp.where(lanev == 1, qsum, 0.0))

    @pl.when(i == NB - 1)
    def _finalize():
        s = sums_ref[...]
        rl = jnp.sum(jnp.where(lanev == 0, s, 0.0)) / (BATCH * IN_DIM)
        qm = jnp.sum(jnp.where(lanev == 1, s, 0.0)) / (BATCH * LATENT)
        ql = qm + BETA * qm
        tot = rl + ql
        sums_ref[...] = (s
                         + jnp.where(lanev == 2, rl, 0.0)
                         + jnp.where(lanev == 3, ql, 0.0)
                         + jnp.where(lanev == 4, tot, 0.0))


def kernel(x, W_e1, b_e1, W_e2, b_e2, codebooks, W_d1, b_d1, W_d2, b_d2):
    cb0 = codebooks[0]
    cb1 = codebooks[1]
    full = lambda shape: pl.BlockSpec(shape, lambda i: tuple(0 for _ in shape))
    recon, idx0, idx1, sums = pl.pallas_call(
        _fused_kernel,
        grid=(NB,),
        in_specs=[
            pl.BlockSpec((BB, IN_DIM), lambda i: (i, 0)),
            full((IN_DIM, HIDDEN)),
            full((1, HIDDEN)),
            full((HIDDEN, LATENT)),
            full((1, LATENT)),
            full((K, LATENT)),
            full((K, LATENT)),
            full((LATENT, HIDDEN)),
            full((1, HIDDEN)),
            full((HIDDEN, IN_DIM)),
            full((1, IN_DIM)),
        ],
        out_specs=[
            pl.BlockSpec((BB, IN_DIM), lambda i: (i, 0)),
            pl.BlockSpec((BB, 1), lambda i: (i, 0)),
            pl.BlockSpec((BB, 1), lambda i: (i, 0)),
            pl.BlockSpec((1, 128), lambda i: (0, 0)),
        ],
        out_shape=[
            jax.ShapeDtypeStruct((BATCH, IN_DIM), jnp.float32),
            jax.ShapeDtypeStruct((BATCH, 1), jnp.int32),
            jax.ShapeDtypeStruct((BATCH, 1), jnp.int32),
            jax.ShapeDtypeStruct((1, 128), jnp.float32),
        ],
    )(x, W_e1, b_e1.reshape(1, -1), W_e2, b_e2.reshape(1, -1), cb0, cb1,
      W_d1, b_d1.reshape(1, -1), W_d2, b_d2.reshape(1, -1))
    indices = jnp.concatenate([idx0, idx1], axis=1)
    recon_loss = sums[0, 2]
    q_loss = sums[0, 3]
    total_loss = sums[0, 4]
    return recon, indices, total_loss, recon_loss, q_loss


# speedup vs baseline: 1.0277x; 1.0277x over previous
import jax
import jax.numpy as jnp
from jax.experimental import pallas as pl

IN_DIM = 72
HIDDEN = 512
LATENT = 256
LEVELS = 2
K = 512
BETA = 0.25
BATCH = 4096
BB = 4096
NB = BATCH // BB
NCH = 1
CB = BB // NCH


def _fused_kernel(x_ref, we1_ref, be1_ref, we2_ref, be2_ref, cb0_ref, cb1_ref,
                  wd1_ref, bd1_ref, wd2_ref, bd2_ref,
                  recon_ref, idx0_ref, idx1_ref, sums_ref):
    i = pl.program_id(0)

    @pl.when(i == 0)
    def _init():
        sums_ref[...] = jnp.zeros_like(sums_ref)

    lanef = jax.lax.broadcasted_iota(
        jnp.int32, (CB, K), 1).astype(jnp.float32)

    def vq_level(residual, cb, exact_q):
        c2 = jnp.sum(cb * cb, axis=1)[None, :]
        r2 = jnp.sum(residual * residual, axis=1, keepdims=True)
        cross = 2.0 * jax.lax.dot_general(
            residual, cb, (((1,), (1,)), ((), ())),
            preferred_element_type=jnp.float32)
        d = (r2 - cross) + c2
        dmin = jnp.min(d, axis=1, keepdims=True)
        idxf = jnp.min(jnp.where(d == dmin, lanef, float(K)), axis=1,
                       keepdims=True)
        idx = idxf.astype(jnp.int32)
        oh = (lanef == idxf).astype(jnp.float32).astype(jnp.bfloat16)
        g = lambda m: jnp.dot(oh, m.astype(jnp.bfloat16),
                              preferred_element_type=jnp.float32)
        if exact_q:
            hi = cb.astype(jnp.bfloat16).astype(jnp.float32)
            r1 = cb - hi
            mid = r1.astype(jnp.bfloat16).astype(jnp.float32)
            lo = r1 - mid
            q = (g(hi) + g(mid)) + g(lo)
        else:
            q = g(cb)
        return idx, q

    rsum = 0.0
    qsum = 0.0
    for c in range(NCH):
        rows = slice(c * CB, (c + 1) * CB)
        x = x_ref[rows, :]
        h = jnp.maximum(
            jnp.dot(x, we1_ref[...], preferred_element_type=jnp.float32)
            + be1_ref[...], 0.0)
        latent = (jnp.dot(h, we2_ref[...], preferred_element_type=jnp.float32)
                  + be2_ref[...])
        idx0, q0 = vq_level(latent, cb0_ref[...], exact_q=True)
        residual = latent - q0
        idx1, q1 = vq_level(residual, cb1_ref[...], exact_q=False)
        quantized = q0 + q1
        qdiff = latent - quantized
        qst = latent + (quantized - latent)
        hd = jnp.maximum(
            jnp.dot(qst, wd1_ref[...], preferred_element_type=jnp.float32)
            + bd1_ref[...], 0.0)
        recon = (jnp.dot(hd, wd2_ref[...], preferred_element_type=jnp.float32)
                 + bd2_ref[...])

        recon_ref[rows, :] = recon
        idx0_ref[rows, :] = idx0
        idx1_ref[rows, :] = idx1
        rsum += jnp.sum((recon - x) ** 2)
        qsum += jnp.sum(qdiff * qdiff)

    lanev = jax.lax.broadcasted_iota(jnp.int32, sums_ref.shape, 1)
    sums_ref[...] += (jnp.where(lanev == 0, rsum, 0.0)
                      + jnp.where(lanev == 1, qsum, 0.0))

    @pl.when(i == NB - 1)
    def _finalize():
        s = sums_ref[...]
        rl = jnp.sum(jnp.where(lanev == 0, s, 0.0)) / (BATCH * IN_DIM)
        qm = jnp.sum(jnp.where(lanev == 1, s, 0.0)) / (BATCH * LATENT)
        ql = qm + BETA * qm
        tot = rl + ql
        sums_ref[...] = (s
                         + jnp.where(lanev == 2, rl, 0.0)
                         + jnp.where(lanev == 3, ql, 0.0)
                         + jnp.where(lanev == 4, tot, 0.0))


def kernel(x, W_e1, b_e1, W_e2, b_e2, codebooks, W_d1, b_d1, W_d2, b_d2):
    cb0 = codebooks[0]
    cb1 = codebooks[1]
    full = lambda shape: pl.BlockSpec(shape, lambda i: tuple(0 for _ in shape))
    recon, idx0, idx1, sums = pl.pallas_call(
        _fused_kernel,
        grid=(NB,),
        in_specs=[
            pl.BlockSpec((BB, IN_DIM), lambda i: (i, 0)),
            full((IN_DIM, HIDDEN)),
            full((1, HIDDEN)),
            full((HIDDEN, LATENT)),
            full((1, LATENT)),
            full((K, LATENT)),
            full((K, LATENT)),
            full((LATENT, HIDDEN)),
            full((1, HIDDEN)),
            full((HIDDEN, IN_DIM)),
            full((1, IN_DIM)),
        ],
        out_specs=[
            pl.BlockSpec((BB, IN_DIM), lambda i: (i, 0)),
            pl.BlockSpec((BB, 1), lambda i: (i, 0)),
            pl.BlockSpec((BB, 1), lambda i: (i, 0)),
            pl.BlockSpec((1, 128), lambda i: (0, 0)),
        ],
        out_shape=[
            jax.ShapeDtypeStruct((BATCH, IN_DIM), jnp.float32),
            jax.ShapeDtypeStruct((BATCH, 1), jnp.int32),
            jax.ShapeDtypeStruct((BATCH, 1), jnp.int32),
            jax.ShapeDtypeStruct((1, 128), jnp.float32),
        ],
    )(x, W_e1, b_e1.reshape(1, -1), W_e2, b_e2.reshape(1, -1), cb0, cb1,
      W_d1, b_d1.reshape(1, -1), W_d2, b_d2.reshape(1, -1))
    indices = jnp.concatenate([idx0, idx1], axis=1)
    recon_loss = sums[0, 2]
    q_loss = sums[0, 3]
    total_loss = sums[0, 4]
    return recon, indices, total_loss, recon_loss, q_loss
